# Initial kernel scaffold; baseline (speedup 1.0000x reference)
import functools

import jax
import jax.numpy as jnp
from jax import lax
from jax.experimental import pallas as pl
from jax.experimental.pallas import tpu as pltpu
from jax.experimental.pallas import tpu_sc as plsc

N = 10000
E = 320000
D = 128

NC = 2
NS = 16
NW = NC * NS
EB = 80
E_PER_TILE = E // NW
N_PER_SUB = N // NS
DEG_W = 16

assert E % NW == 0 and E_PER_TILE % EB == 0
assert N % NS == 0 and EB % 8 == 0 and E_PER_TILE % 8 == 0

_MESH = plsc.VectorSubcoreMesh(core_axis_name="c", subcore_axis_name="s")


def _deg_body(col_hbm, ones_hbm, zeros_hbm, out_hbm, idx_v, ones_v, acc_sh):
    c = lax.axis_index("c")
    s = lax.axis_index("s")
    wid = s * NC + c
    nslc = pl.ds(s * N_PER_SUB, N_PER_SUB)
    pltpu.sync_copy(zeros_hbm.at[nslc], acc_sh.at[nslc])
    pltpu.sync_copy(ones_hbm, ones_v)
    plsc.subcore_barrier()
    base = wid * E_PER_TILE

    def step(i, carry):
        off = pl.multiple_of(base + i * EB, EB)
        pltpu.sync_copy(col_hbm.at[pl.ds(off, EB)], idx_v)
        pltpu.sync_copy(ones_v, acc_sh.at[idx_v], add=True)
        return carry

    lax.fori_loop(0, E_PER_TILE // EB, step, 0)
    plsc.subcore_barrier()
    pltpu.sync_copy(acc_sh.at[nslc], out_hbm.at[c, nslc])


_deg = pl.kernel(
    _deg_body,
    out_type=jax.ShapeDtypeStruct((NC, N, DEG_W), jnp.float32),
    mesh=_MESH,
    scratch_types=[
        pltpu.VMEM((EB,), jnp.int32),
        pltpu.VMEM((EB, DEG_W), jnp.float32),
        pltpu.VMEM_SHARED((N, DEG_W), jnp.float32),
    ],
)


def _msg_body(y_hbm, row_hbm, col_hbm, zeros_hbm, out_hbm,
              idxr_v, idxc_v, rows_v, acc_sh, gsem):
    c = lax.axis_index("c")
    s = lax.axis_index("s")
    wid = s * NC + c
    nslc = pl.ds(s * N_PER_SUB, N_PER_SUB)
    pltpu.sync_copy(zeros_hbm.at[nslc], acc_sh.at[nslc])
    plsc.subcore_barrier()
    base = wid * E_PER_TILE

    def step(i, carry):
        off = pl.multiple_of(base + i * EB, EB)
        pltpu.sync_copy(row_hbm.at[pl.ds(off, EB)], idxr_v)
        pltpu.sync_copy(col_hbm.at[pl.ds(off, EB)], idxc_v)
        pltpu.async_copy(y_hbm.at[idxr_v], rows_v, gsem).wait()
        pltpu.sync_copy(rows_v, acc_sh.at[idxc_v], add=True)
        return carry

    lax.fori_loop(0, E_PER_TILE // EB, step, 0)
    plsc.subcore_barrier()
    pltpu.sync_copy(acc_sh.at[nslc], out_hbm.at[c, nslc])


_msg = pl.kernel(
    _msg_body,
    out_type=jax.ShapeDtypeStruct((NC, N, D), jnp.float32),
    mesh=_MESH,
    scratch_types=[
        pltpu.VMEM((EB,), jnp.int32),
        pltpu.VMEM((EB,), jnp.int32),
        pltpu.VMEM((EB, D), jnp.float32),
        pltpu.VMEM_SHARED((N, D), jnp.float32),
        pltpu.SemaphoreType.DMA,
    ],
)

BN = 1000


def _dinv_of(degp):
    deg = jnp.sum(degp, axis=(0, 2)) * (1.0 / DEG_W) + 1.0
    return lax.rsqrt(deg)


def _tc1_body(x_ref, wt_ref, degp_ref, y_ref):
    dinv = _dinv_of(degp_ref[...])
    z = jnp.dot(x_ref[...], wt_ref[...], preferred_element_type=jnp.float32)
    y_ref[...] = z * dinv[:, None]


_tc1 = pl.pallas_call(
    _tc1_body,
    grid=(N // BN,),
    in_specs=[
        pl.BlockSpec((BN, D), lambda i: (i, 0)),
        pl.BlockSpec((D, D), lambda i: (0, 0)),
        pl.BlockSpec((NC, BN, DEG_W), lambda i: (0, i, 0)),
    ],
    out_specs=pl.BlockSpec((BN, D), lambda i: (i, 0)),
    out_shape=jax.ShapeDtypeStruct((N, D), jnp.float32),
)


def _tc2_body(y_ref, p_ref, degp_ref, wt_ref, b_ref, o_ref):
    dinv = _dinv_of(degp_ref[...])
    ssum = p_ref[0] + p_ref[1] + y_ref[...]
    h = jnp.maximum(ssum * dinv[:, None] + b_ref[...], 0.0)
    z = jnp.dot(h, wt_ref[...], preferred_element_type=jnp.float32)
    o_ref[...] = z * dinv[:, None]


_tc2 = pl.pallas_call(
    _tc2_body,
    grid=(N // BN,),
    in_specs=[
        pl.BlockSpec((BN, D), lambda i: (i, 0)),
        pl.BlockSpec((NC, BN, D), lambda i: (0, i, 0)),
        pl.BlockSpec((NC, BN, DEG_W), lambda i: (0, i, 0)),
        pl.BlockSpec((D, D), lambda i: (0, 0)),
        pl.BlockSpec((1, D), lambda i: (0, 0)),
    ],
    out_specs=pl.BlockSpec((BN, D), lambda i: (i, 0)),
    out_shape=jax.ShapeDtypeStruct((N, D), jnp.float32),
)


def _tc3_body(y_ref, p_ref, degp_ref, b_ref, o_ref):
    dinv = _dinv_of(degp_ref[...])
    ssum = p_ref[0] + p_ref[1] + y_ref[...]
    o_ref[...] = jnp.maximum(ssum * dinv[:, None] + b_ref[...], 0.0)


_tc3 = pl.pallas_call(
    _tc3_body,
    grid=(N // BN,),
    in_specs=[
        pl.BlockSpec((BN, D), lambda i: (i, 0)),
        pl.BlockSpec((NC, BN, D), lambda i: (0, i, 0)),
        pl.BlockSpec((NC, BN, DEG_W), lambda i: (0, i, 0)),
        pl.BlockSpec((1, D), lambda i: (0, 0)),
    ],
    out_specs=pl.BlockSpec((BN, D), lambda i: (i, 0)),
    out_shape=jax.ShapeDtypeStruct((N, D), jnp.float32),
)


def kernel(x, edge_index, W1, b1, W2, b2):
    row = edge_index[0]
    col = edge_index[1]
    ones = jnp.ones((EB, DEG_W), jnp.float32)
    z16 = jnp.zeros((N, DEG_W), jnp.float32)
    z128 = jnp.zeros((N, D), jnp.float32)

    degp = _deg(col, ones, z16)
    y1 = _tc1(x, W1.T, degp)
    p1 = _msg(y1, row, col, z128)
    y2 = _tc2(y1, p1, degp, W2.T, b1.reshape(1, D))
    p2 = _msg(y2, row, col, z128)
    return _tc3(y2, p2, degp, b2.reshape(1, D))


# speedup vs baseline: 12.2683x; 12.2683x over previous
import functools

import jax
import jax.numpy as jnp
from jax import lax
from jax.experimental import pallas as pl
from jax.experimental.pallas import tpu as pltpu
from jax.experimental.pallas import tpu_sc as plsc

N = 10000
E = 320000
D = 128

NC = 2
NS = 16
NW = NC * NS
EB = 80
E_PER_TILE = E // NW
NP = 10240
N_PER_SUB = NP // NS
DEG_W = 128

assert E % NW == 0 and E_PER_TILE % EB == 0
assert NP % (8 * NS) == 0 and EB % 8 == 0 and E_PER_TILE % 8 == 0

def _deg_body(col_hbm, ones_hbm, zeros_hbm, out_hbm, idx_v, ones_v, acc_sh):
    c = lax.axis_index("c")
    s = lax.axis_index("s")
    wid = s * NC + c
    nslc = pl.ds(s * N_PER_SUB, N_PER_SUB)
    pltpu.sync_copy(zeros_hbm.at[nslc], acc_sh.at[nslc])
    pltpu.sync_copy(ones_hbm, ones_v)
    plsc.subcore_barrier()
    base = wid * E_PER_TILE

    def step(i, carry):
        off = pl.multiple_of(base + i * EB, EB)
        pltpu.sync_copy(col_hbm.at[pl.ds(off, EB)], idx_v)
        pltpu.sync_copy(ones_v, acc_sh.at[idx_v], add=True)
        return carry

    lax.fori_loop(0, E_PER_TILE // EB, step, 0)
    plsc.subcore_barrier()
    pltpu.sync_copy(acc_sh.at[nslc], out_hbm.at[c, nslc])


@functools.cache
def _get_deg():
    return pl.kernel(
        _deg_body,
        out_type=jax.ShapeDtypeStruct((NC, NP, DEG_W), jnp.float32),
        mesh=plsc.VectorSubcoreMesh(core_axis_name="c", subcore_axis_name="s"),
        scratch_types=[
            pltpu.VMEM((EB,), jnp.int32),
            pltpu.VMEM((EB, DEG_W), jnp.float32),
            pltpu.VMEM_SHARED((NP, DEG_W), jnp.float32),
        ],
    )


def _msg_body(y_hbm, row_hbm, col_hbm, zeros_hbm, out_hbm,
              idxr_v, idxc_v, rows_v, acc_sh, gsem):
    c = lax.axis_index("c")
    s = lax.axis_index("s")
    wid = s * NC + c
    nslc = pl.ds(s * N_PER_SUB, N_PER_SUB)
    pltpu.sync_copy(zeros_hbm.at[nslc], acc_sh.at[nslc])
    plsc.subcore_barrier()
    base = wid * E_PER_TILE

    def step(i, carry):
        off = pl.multiple_of(base + i * EB, EB)
        pltpu.sync_copy(row_hbm.at[pl.ds(off, EB)], idxr_v)
        pltpu.sync_copy(col_hbm.at[pl.ds(off, EB)], idxc_v)
        pltpu.async_copy(y_hbm.at[idxr_v], rows_v, gsem).wait()
        pltpu.sync_copy(rows_v, acc_sh.at[idxc_v], add=True)
        return carry

    lax.fori_loop(0, E_PER_TILE // EB, step, 0)
    plsc.subcore_barrier()
    pltpu.sync_copy(acc_sh.at[nslc], out_hbm.at[c, nslc])


NBATCH = E_PER_TILE // EB


def _msg2_body(y_hbm, row_hbm, col_hbm, zeros_hbm, out_hbm,
               idxr_v, idxc_v, rows_v, acc_sh, gsem0, gsem1, csem0, csem1):
    c = lax.axis_index("c")
    s = lax.axis_index("s")
    wid = s * NC + c
    nslc = pl.ds(s * N_PER_SUB, N_PER_SUB)
    base = pl.multiple_of(wid * E_PER_TILE, EB)
    pltpu.sync_copy(row_hbm.at[pl.ds(base, E_PER_TILE)], idxr_v)
    pltpu.sync_copy(zeros_hbm.at[nslc], acc_sh.at[nslc])
    plsc.subcore_barrier()

    gsems = (gsem0, gsem1)
    csems = (csem0, csem1)

    def start_batch(i, b):
        off = pl.multiple_of(base + i * EB, EB)
        pltpu.async_copy(col_hbm.at[pl.ds(off, EB)], idxc_v.at[b], csems[b])
        pltpu.async_copy(y_hbm.at[idxr_v.at[pl.ds(i * EB, EB)]],
                         rows_v.at[b], gsems[b])

    def finish_batch(i, b):
        pltpu.make_async_copy(col_hbm.at[pl.ds(0, EB)], idxc_v.at[b],
                              csems[b]).wait()
        pltpu.make_async_copy(y_hbm.at[pl.ds(0, EB)], rows_v.at[b],
                              gsems[b]).wait()
        pltpu.sync_copy(rows_v.at[b], acc_sh.at[idxc_v.at[b]], add=True)

    def _do(i, b):
        @pl.when(i + 1 < NBATCH)
        def _():
            start_batch(i + 1, 1 - b)

        finish_batch(i, b)

    start_batch(0, 0)

    def step(i, carry):
        @pl.when(lax.rem(i, 2) == 0)
        def _():
            _do(i, 0)

        @pl.when(lax.rem(i, 2) == 1)
        def _():
            _do(i, 1)

        return carry

    lax.fori_loop(0, NBATCH, step, 0)
    plsc.subcore_barrier()
    pltpu.sync_copy(acc_sh.at[nslc], out_hbm.at[c, nslc])


@functools.cache
def _get_msg2():
    return pl.kernel(
        _msg2_body,
        out_type=jax.ShapeDtypeStruct((NC, NP, D), jnp.float32),
        mesh=plsc.VectorSubcoreMesh(core_axis_name="c", subcore_axis_name="s"),
        scratch_types=[
            pltpu.VMEM((E_PER_TILE,), jnp.int32),
            pltpu.VMEM((2, EB), jnp.int32),
            pltpu.VMEM((2, EB, D), jnp.float32),
            pltpu.VMEM_SHARED((NP, D), jnp.float32),
            pltpu.SemaphoreType.DMA,
            pltpu.SemaphoreType.DMA,
            pltpu.SemaphoreType.DMA,
            pltpu.SemaphoreType.DMA,
        ],
    )


@functools.cache
def _get_msg():
    return pl.kernel(
        _msg_body,
        out_type=jax.ShapeDtypeStruct((NC, NP, D), jnp.float32),
        mesh=plsc.VectorSubcoreMesh(core_axis_name="c", subcore_axis_name="s"),
        scratch_types=[
            pltpu.VMEM((EB,), jnp.int32),
            pltpu.VMEM((EB,), jnp.int32),
            pltpu.VMEM((EB, D), jnp.float32),
            pltpu.VMEM_SHARED((NP, D), jnp.float32),
            pltpu.SemaphoreType.DMA,
        ],
    )

BN = 1000


def _dinv_of(degp):
    deg = jnp.sum(degp, axis=(0, 2)) * (1.0 / DEG_W) + 1.0
    return lax.rsqrt(deg)


def _tc1_body(x_ref, wt_ref, degp_ref, y_ref):
    dinv = _dinv_of(degp_ref[...])
    z = jnp.dot(x_ref[...], wt_ref[...], preferred_element_type=jnp.float32)
    y_ref[...] = z * dinv[:, None]


_tc1 = pl.pallas_call(
    _tc1_body,
    grid=(N // BN,),
    in_specs=[
        pl.BlockSpec((BN, D), lambda i: (i, 0)),
        pl.BlockSpec((D, D), lambda i: (0, 0)),
        pl.BlockSpec((NC, BN, DEG_W), lambda i: (0, i, 0)),
    ],
    out_specs=pl.BlockSpec((BN, D), lambda i: (i, 0)),
    out_shape=jax.ShapeDtypeStruct((N, D), jnp.float32),
)


def _tc2_body(y_ref, p_ref, degp_ref, wt_ref, b_ref, o_ref):
    dinv = _dinv_of(degp_ref[...])
    ssum = p_ref[0] + p_ref[1] + y_ref[...]
    h = jnp.maximum(ssum * dinv[:, None] + b_ref[...], 0.0)
    z = jnp.dot(h, wt_ref[...], preferred_element_type=jnp.float32)
    o_ref[...] = z * dinv[:, None]


_tc2 = pl.pallas_call(
    _tc2_body,
    grid=(N // BN,),
    in_specs=[
        pl.BlockSpec((BN, D), lambda i: (i, 0)),
        pl.BlockSpec((NC, BN, D), lambda i: (0, i, 0)),
        pl.BlockSpec((NC, BN, DEG_W), lambda i: (0, i, 0)),
        pl.BlockSpec((D, D), lambda i: (0, 0)),
        pl.BlockSpec((1, D), lambda i: (0, 0)),
    ],
    out_specs=pl.BlockSpec((BN, D), lambda i: (i, 0)),
    out_shape=jax.ShapeDtypeStruct((N, D), jnp.float32),
)


def _tc3_body(y_ref, p_ref, degp_ref, b_ref, o_ref):
    dinv = _dinv_of(degp_ref[...])
    ssum = p_ref[0] + p_ref[1] + y_ref[...]
    o_ref[...] = jnp.maximum(ssum * dinv[:, None] + b_ref[...], 0.0)


_tc3 = pl.pallas_call(
    _tc3_body,
    grid=(N // BN,),
    in_specs=[
        pl.BlockSpec((BN, D), lambda i: (i, 0)),
        pl.BlockSpec((NC, BN, D), lambda i: (0, i, 0)),
        pl.BlockSpec((NC, BN, DEG_W), lambda i: (0, i, 0)),
        pl.BlockSpec((1, D), lambda i: (0, 0)),
    ],
    out_specs=pl.BlockSpec((BN, D), lambda i: (i, 0)),
    out_shape=jax.ShapeDtypeStruct((N, D), jnp.float32),
)


def kernel(x, edge_index, W1, b1, W2, b2):
    row = edge_index[0]
    col = edge_index[1]
    z128 = jnp.zeros((NP, D), jnp.float32)
    ones = jnp.ones((EB, DEG_W), jnp.float32)

    deg_k, msg_k = _get_deg(), _get_msg()
    degp = deg_k(col, ones, z128)
    y1 = _tc1(x, W1.T, degp)
    p1 = msg_k(y1, row, col, z128)
    y2 = _tc2(y1, p1, degp, W2.T, b1.reshape(1, D))
    p2 = msg_k(y2, row, col, z128)
    return _tc3(y2, p2, degp, b2.reshape(1, D))


# speedup vs baseline: 26.1703x; 2.1332x over previous
import functools

import jax
import jax.numpy as jnp
from jax import lax
from jax.experimental import pallas as pl
from jax.experimental.pallas import tpu as pltpu
from jax.experimental.pallas import tpu_sc as plsc

N = 10000
E = 320000
D = 128

NC = 2
NS = 16
NW = NC * NS
EB = 80
E_PER_TILE = E // NW
NP = 10240
N_PER_SUB = NP // NS
DEG_W = 128

assert E % NW == 0 and E_PER_TILE % EB == 0
assert NP % (8 * NS) == 0 and EB % 8 == 0 and E_PER_TILE % 8 == 0

def _deg_body(col_hbm, ones_hbm, zeros_hbm, out_hbm, idx_v, ones_v, acc_sh):
    c = lax.axis_index("c")
    s = lax.axis_index("s")
    wid = s * NC + c
    nslc = pl.ds(s * N_PER_SUB, N_PER_SUB)
    pltpu.sync_copy(zeros_hbm.at[nslc], acc_sh.at[nslc])
    pltpu.sync_copy(ones_hbm, ones_v)
    plsc.subcore_barrier()
    base = wid * E_PER_TILE

    def step(i, carry):
        off = pl.multiple_of(base + i * EB, EB)
        pltpu.sync_copy(col_hbm.at[pl.ds(off, EB)], idx_v)
        pltpu.sync_copy(ones_v, acc_sh.at[idx_v], add=True)
        return carry

    lax.fori_loop(0, E_PER_TILE // EB, step, 0)
    plsc.subcore_barrier()
    pltpu.sync_copy(acc_sh.at[nslc], out_hbm.at[c, nslc])


@functools.cache
def _get_deg():
    return pl.kernel(
        _deg_body,
        out_type=jax.ShapeDtypeStruct((NC, NP, DEG_W), jnp.float32),
        mesh=plsc.VectorSubcoreMesh(core_axis_name="c", subcore_axis_name="s"),
        scratch_types=[
            pltpu.VMEM((EB,), jnp.int32),
            pltpu.VMEM((EB, DEG_W), jnp.float32),
            pltpu.VMEM_SHARED((NP, DEG_W), jnp.float32),
        ],
    )


def _msg_body(y_hbm, row_hbm, col_hbm, zeros_hbm, out_hbm,
              idxr_v, idxc_v, rows_v, acc_sh, gsem):
    c = lax.axis_index("c")
    s = lax.axis_index("s")
    wid = s * NC + c
    nslc = pl.ds(s * N_PER_SUB, N_PER_SUB)
    pltpu.sync_copy(zeros_hbm.at[nslc], acc_sh.at[nslc])
    plsc.subcore_barrier()
    base = wid * E_PER_TILE

    def step(i, carry):
        off = pl.multiple_of(base + i * EB, EB)
        pltpu.sync_copy(row_hbm.at[pl.ds(off, EB)], idxr_v)
        pltpu.sync_copy(col_hbm.at[pl.ds(off, EB)], idxc_v)
        pltpu.async_copy(y_hbm.at[idxr_v], rows_v, gsem).wait()
        pltpu.sync_copy(rows_v, acc_sh.at[idxc_v], add=True)
        return carry

    lax.fori_loop(0, E_PER_TILE // EB, step, 0)
    plsc.subcore_barrier()
    pltpu.sync_copy(acc_sh.at[nslc], out_hbm.at[c, nslc])


NBATCH = E_PER_TILE // EB


def _deg2_body(col_hbm, ones_hbm, zeros_hbm, out_hbm,
               idxc_v, ones_v, acc_sh, csem0, csem1):
    c = lax.axis_index("c")
    s = lax.axis_index("s")
    wid = s * NC + c
    nslc = pl.ds(s * N_PER_SUB, N_PER_SUB)
    base = pl.multiple_of(wid * E_PER_TILE, EB)
    pltpu.sync_copy(zeros_hbm.at[nslc], acc_sh.at[nslc])
    pltpu.sync_copy(ones_hbm, ones_v)
    plsc.subcore_barrier()

    csems = (csem0, csem1)

    def start_batch(i, b):
        off = pl.multiple_of(base + i * EB, EB)
        pltpu.async_copy(col_hbm.at[pl.ds(off, EB)], idxc_v.at[b], csems[b])

    def finish_batch(i, b):
        pltpu.make_async_copy(col_hbm.at[pl.ds(0, EB)], idxc_v.at[b],
                              csems[b]).wait()
        pltpu.sync_copy(ones_v, acc_sh.at[idxc_v.at[b]], add=True)

    def _do(i, b):
        @pl.when(i + 1 < NBATCH)
        def _():
            start_batch(i + 1, 1 - b)

        finish_batch(i, b)

    start_batch(0, 0)

    def step(i, carry):
        @pl.when(lax.rem(i, 2) == 0)
        def _():
            _do(i, 0)

        @pl.when(lax.rem(i, 2) == 1)
        def _():
            _do(i, 1)

        return carry

    lax.fori_loop(0, NBATCH, step, 0)
    plsc.subcore_barrier()
    pltpu.sync_copy(acc_sh.at[nslc], out_hbm.at[c, nslc])


@functools.cache
def _get_deg2():
    return pl.kernel(
        _deg2_body,
        out_type=jax.ShapeDtypeStruct((NC, NP, DEG_W), jnp.float32),
        mesh=plsc.VectorSubcoreMesh(core_axis_name="c", subcore_axis_name="s"),
        scratch_types=[
            pltpu.VMEM((2, EB), jnp.int32),
            pltpu.VMEM((EB, DEG_W), jnp.float32),
            pltpu.VMEM_SHARED((NP, DEG_W), jnp.float32),
            pltpu.SemaphoreType.DMA,
            pltpu.SemaphoreType.DMA,
        ],
    )


def _msg2_body(y_hbm, row_hbm, col_hbm, zeros_hbm, out_hbm,
               idxr_v, idxc_v, rows_v, acc_sh, gsem0, gsem1, csem0, csem1):
    c = lax.axis_index("c")
    s = lax.axis_index("s")
    wid = s * NC + c
    nslc = pl.ds(s * N_PER_SUB, N_PER_SUB)
    base = pl.multiple_of(wid * E_PER_TILE, EB)
    pltpu.sync_copy(row_hbm.at[pl.ds(base, E_PER_TILE)], idxr_v)
    pltpu.sync_copy(zeros_hbm.at[nslc], acc_sh.at[nslc])
    plsc.subcore_barrier()

    gsems = (gsem0, gsem1)
    csems = (csem0, csem1)

    def start_batch(i, b):
        off = pl.multiple_of(base + i * EB, EB)
        pltpu.async_copy(col_hbm.at[pl.ds(off, EB)], idxc_v.at[b], csems[b])
        pltpu.async_copy(y_hbm.at[idxr_v.at[pl.ds(i * EB, EB)]],
                         rows_v.at[b], gsems[b])

    def finish_batch(i, b):
        pltpu.make_async_copy(col_hbm.at[pl.ds(0, EB)], idxc_v.at[b],
                              csems[b]).wait()
        pltpu.make_async_copy(y_hbm.at[pl.ds(0, EB)], rows_v.at[b],
                              gsems[b]).wait()
        pltpu.sync_copy(rows_v.at[b], acc_sh.at[idxc_v.at[b]], add=True)

    def _do(i, b):
        @pl.when(i + 1 < NBATCH)
        def _():
            start_batch(i + 1, 1 - b)

        finish_batch(i, b)

    start_batch(0, 0)

    def step(i, carry):
        @pl.when(lax.rem(i, 2) == 0)
        def _():
            _do(i, 0)

        @pl.when(lax.rem(i, 2) == 1)
        def _():
            _do(i, 1)

        return carry

    lax.fori_loop(0, NBATCH, step, 0)
    plsc.subcore_barrier()
    pltpu.sync_copy(acc_sh.at[nslc], out_hbm.at[c, nslc])


@functools.cache
def _get_msg2():
    return pl.kernel(
        _msg2_body,
        out_type=jax.ShapeDtypeStruct((NC, NP, D), jnp.float32),
        mesh=plsc.VectorSubcoreMesh(core_axis_name="c", subcore_axis_name="s"),
        scratch_types=[
            pltpu.VMEM((E_PER_TILE,), jnp.int32),
            pltpu.VMEM((2, EB), jnp.int32),
            pltpu.VMEM((2, EB, D), jnp.float32),
            pltpu.VMEM_SHARED((NP, D), jnp.float32),
            pltpu.SemaphoreType.DMA,
            pltpu.SemaphoreType.DMA,
            pltpu.SemaphoreType.DMA,
            pltpu.SemaphoreType.DMA,
        ],
    )


@functools.cache
def _get_msg():
    return pl.kernel(
        _msg_body,
        out_type=jax.ShapeDtypeStruct((NC, NP, D), jnp.float32),
        mesh=plsc.VectorSubcoreMesh(core_axis_name="c", subcore_axis_name="s"),
        scratch_types=[
            pltpu.VMEM((EB,), jnp.int32),
            pltpu.VMEM((EB,), jnp.int32),
            pltpu.VMEM((EB, D), jnp.float32),
            pltpu.VMEM_SHARED((NP, D), jnp.float32),
            pltpu.SemaphoreType.DMA,
        ],
    )

BN = 1000


def _dinv_of(degp):
    deg = jnp.sum(degp, axis=(0, 2)) * (1.0 / DEG_W) + 1.0
    return lax.rsqrt(deg)


def _tc1_body(x_ref, wt_ref, degp_ref, y_ref):
    dinv = _dinv_of(degp_ref[...])
    z = jnp.dot(x_ref[...], wt_ref[...], preferred_element_type=jnp.float32)
    y_ref[...] = z * dinv[:, None]


_tc1 = pl.pallas_call(
    _tc1_body,
    grid=(N // BN,),
    in_specs=[
        pl.BlockSpec((BN, D), lambda i: (i, 0)),
        pl.BlockSpec((D, D), lambda i: (0, 0)),
        pl.BlockSpec((NC, BN, DEG_W), lambda i: (0, i, 0)),
    ],
    out_specs=pl.BlockSpec((BN, D), lambda i: (i, 0)),
    out_shape=jax.ShapeDtypeStruct((N, D), jnp.float32),
)


def _tc2_body(y_ref, p_ref, degp_ref, wt_ref, b_ref, o_ref):
    dinv = _dinv_of(degp_ref[...])
    ssum = p_ref[0] + p_ref[1] + y_ref[...]
    h = jnp.maximum(ssum * dinv[:, None] + b_ref[...], 0.0)
    z = jnp.dot(h, wt_ref[...], preferred_element_type=jnp.float32)
    o_ref[...] = z * dinv[:, None]


_tc2 = pl.pallas_call(
    _tc2_body,
    grid=(N // BN,),
    in_specs=[
        pl.BlockSpec((BN, D), lambda i: (i, 0)),
        pl.BlockSpec((NC, BN, D), lambda i: (0, i, 0)),
        pl.BlockSpec((NC, BN, DEG_W), lambda i: (0, i, 0)),
        pl.BlockSpec((D, D), lambda i: (0, 0)),
        pl.BlockSpec((1, D), lambda i: (0, 0)),
    ],
    out_specs=pl.BlockSpec((BN, D), lambda i: (i, 0)),
    out_shape=jax.ShapeDtypeStruct((N, D), jnp.float32),
)


def _tc3_body(y_ref, p_ref, degp_ref, b_ref, o_ref):
    dinv = _dinv_of(degp_ref[...])
    ssum = p_ref[0] + p_ref[1] + y_ref[...]
    o_ref[...] = jnp.maximum(ssum * dinv[:, None] + b_ref[...], 0.0)


_tc3 = pl.pallas_call(
    _tc3_body,
    grid=(N // BN,),
    in_specs=[
        pl.BlockSpec((BN, D), lambda i: (i, 0)),
        pl.BlockSpec((NC, BN, D), lambda i: (0, i, 0)),
        pl.BlockSpec((NC, BN, DEG_W), lambda i: (0, i, 0)),
        pl.BlockSpec((1, D), lambda i: (0, 0)),
    ],
    out_specs=pl.BlockSpec((BN, D), lambda i: (i, 0)),
    out_shape=jax.ShapeDtypeStruct((N, D), jnp.float32),
)


def kernel(x, edge_index, W1, b1, W2, b2):
    row = edge_index[0]
    col = edge_index[1]
    z128 = jnp.zeros((NP, D), jnp.float32)
    ones = jnp.ones((EB, DEG_W), jnp.float32)

    deg_k, msg_k = _get_deg2(), _get_msg2()
    degp = deg_k(col, ones, z128)
    y1 = _tc1(x, W1.T, degp)
    p1 = msg_k(y1, row, col, z128)
    y2 = _tc2(y1, p1, degp, W2.T, b1.reshape(1, D))
    p2 = msg_k(y2, row, col, z128)
    return _tc3(y2, p2, degp, b2.reshape(1, D))


# speedup vs baseline: 29.7636x; 1.1373x over previous
import functools

import jax
import jax.numpy as jnp
from jax import lax
from jax.experimental import pallas as pl
from jax.experimental.pallas import tpu as pltpu
from jax.experimental.pallas import tpu_sc as plsc

N = 10000
E = 320000
D = 128

NC = 2
NS = 16
NW = NC * NS
EB = 80
E_PER_TILE = E // NW
NP = 10240
N_PER_SUB = NP // NS
DEG_W = 128

assert E % NW == 0 and E_PER_TILE % EB == 0
assert NP % (8 * NS) == 0 and EB % 8 == 0 and E_PER_TILE % 8 == 0

NBATCH = E_PER_TILE // EB


def _deg2_body(col_hbm, ones_hbm, zeros_hbm, out_hbm,
               idxc_v, ones_v, acc_sh, csem0, csem1):
    c = lax.axis_index("c")
    s = lax.axis_index("s")
    wid = s * NC + c
    nslc = pl.ds(s * N_PER_SUB, N_PER_SUB)
    base = pl.multiple_of(wid * E_PER_TILE, EB)
    pltpu.sync_copy(zeros_hbm.at[nslc], acc_sh.at[nslc])
    pltpu.sync_copy(ones_hbm, ones_v)
    plsc.subcore_barrier()

    csems = (csem0, csem1)

    def start_batch(i, b):
        off = pl.multiple_of(base + i * EB, EB)
        pltpu.async_copy(col_hbm.at[pl.ds(off, EB)], idxc_v.at[b], csems[b])

    def finish_batch(i, b):
        pltpu.make_async_copy(col_hbm.at[pl.ds(0, EB)], idxc_v.at[b],
                              csems[b]).wait()
        pltpu.sync_copy(ones_v, acc_sh.at[idxc_v.at[b]], add=True)

    def _do(i, b):
        @pl.when(i + 1 < NBATCH)
        def _():
            start_batch(i + 1, 1 - b)

        finish_batch(i, b)

    start_batch(0, 0)

    def step(i, carry):
        @pl.when(lax.rem(i, 2) == 0)
        def _():
            _do(i, 0)

        @pl.when(lax.rem(i, 2) == 1)
        def _():
            _do(i, 1)

        return carry

    lax.fori_loop(0, NBATCH, step, 0)
    plsc.subcore_barrier()
    pltpu.sync_copy(acc_sh.at[nslc], out_hbm.at[c, nslc])


@functools.cache
def _get_deg2():
    return pl.kernel(
        _deg2_body,
        out_type=jax.ShapeDtypeStruct((NC, NP, DEG_W), jnp.float32),
        mesh=plsc.VectorSubcoreMesh(core_axis_name="c", subcore_axis_name="s"),
        scratch_types=[
            pltpu.VMEM((2, EB), jnp.int32),
            pltpu.VMEM((EB, DEG_W), jnp.float32),
            pltpu.VMEM_SHARED((NP, DEG_W), jnp.float32),
            pltpu.SemaphoreType.DMA,
            pltpu.SemaphoreType.DMA,
        ],
    )


NBUF = 3


def _msg2_body(y_hbm, row_hbm, col_hbm, zeros_hbm, out_hbm,
               idxr_v, idxc_v, rows_v, acc_sh, *sems):
    gsems = sems[:NBUF]
    csems = sems[NBUF:]
    c = lax.axis_index("c")
    s = lax.axis_index("s")
    wid = s * NC + c
    nslc = pl.ds(s * N_PER_SUB, N_PER_SUB)
    base = pl.multiple_of(wid * E_PER_TILE, EB)
    pltpu.sync_copy(row_hbm.at[pl.ds(base, E_PER_TILE)], idxr_v)
    pltpu.sync_copy(zeros_hbm.at[nslc], acc_sh.at[nslc])
    plsc.subcore_barrier()

    def start_batch(i, b):
        off = pl.multiple_of(base + i * EB, EB)
        pltpu.async_copy(col_hbm.at[pl.ds(off, EB)], idxc_v.at[b], csems[b])
        pltpu.async_copy(y_hbm.at[idxr_v.at[pl.ds(i * EB, EB)]],
                         rows_v.at[b], gsems[b])

    def finish_batch(i, b):
        pltpu.make_async_copy(col_hbm.at[pl.ds(0, EB)], idxc_v.at[b],
                              csems[b]).wait()
        pltpu.make_async_copy(y_hbm.at[pl.ds(0, EB)], rows_v.at[b],
                              gsems[b]).wait()
        pltpu.sync_copy(rows_v.at[b], acc_sh.at[idxc_v.at[b]], add=True)

    def _do(i, b):
        @pl.when(i + NBUF - 1 < NBATCH)
        def _():
            start_batch(i + NBUF - 1, (b + NBUF - 1) % NBUF)

        finish_batch(i, b)

    for j in range(NBUF - 1):
        start_batch(j, j)

    def step(i, carry):
        for b in range(NBUF):
            @pl.when(lax.rem(i, NBUF) == b)
            def _(b=b):
                _do(i, b)

        return carry

    lax.fori_loop(0, NBATCH, step, 0)
    plsc.subcore_barrier()
    pltpu.sync_copy(acc_sh.at[nslc], out_hbm.at[c, nslc])


@functools.cache
def _get_msg2():
    return pl.kernel(
        _msg2_body,
        out_type=jax.ShapeDtypeStruct((NC, NP, D), jnp.float32),
        mesh=plsc.VectorSubcoreMesh(core_axis_name="c", subcore_axis_name="s"),
        scratch_types=[
            pltpu.VMEM((E_PER_TILE,), jnp.int32),
            pltpu.VMEM((NBUF, EB), jnp.int32),
            pltpu.VMEM((NBUF, EB, D), jnp.float32),
            pltpu.VMEM_SHARED((NP, D), jnp.float32),
        ] + [pltpu.SemaphoreType.DMA] * (2 * NBUF),
    )


BN = 1000


def _dinv_of(degp):
    deg = jnp.sum(degp, axis=(0, 2)) * (1.0 / DEG_W) + 1.0
    return lax.rsqrt(deg)


DINV_W = 16


def _dinv16_of(dref):
    return jnp.sum(dref, axis=1) * (1.0 / DINV_W)


def _tc_mm_body(x_ref, wt_ref, z_ref):
    z_ref[...] = jnp.dot(x_ref[...], wt_ref[...],
                         preferred_element_type=jnp.float32)


_tc_mm = pl.pallas_call(
    _tc_mm_body,
    grid=(N // BN,),
    in_specs=[
        pl.BlockSpec((BN, D), lambda i: (i, 0)),
        pl.BlockSpec((D, D), lambda i: (0, 0)),
    ],
    out_specs=pl.BlockSpec((BN, D), lambda i: (i, 0)),
    out_shape=jax.ShapeDtypeStruct((N, D), jnp.float32),
)


def _tc_scale_body(z_ref, degp_ref, y_ref, dinv_ref):
    dinv = _dinv_of(degp_ref[...])
    y_ref[...] = z_ref[...] * dinv[:, None]
    dinv_ref[...] = jnp.broadcast_to(dinv[:, None], (BN, DINV_W))


_tc_scale = pl.pallas_call(
    _tc_scale_body,
    grid=(N // BN,),
    in_specs=[
        pl.BlockSpec((BN, D), lambda i: (i, 0)),
        pl.BlockSpec((NC, BN, DEG_W), lambda i: (0, i, 0)),
    ],
    out_specs=[
        pl.BlockSpec((BN, D), lambda i: (i, 0)),
        pl.BlockSpec((BN, DINV_W), lambda i: (i, 0)),
    ],
    out_shape=[
        jax.ShapeDtypeStruct((N, D), jnp.float32),
        jax.ShapeDtypeStruct((N, DINV_W), jnp.float32),
    ],
)


def _tc2_body(y_ref, p_ref, dinv16_ref, wt_ref, b_ref, o_ref):
    dinv = _dinv16_of(dinv16_ref[...])
    ssum = p_ref[0] + p_ref[1] + y_ref[...]
    h = jnp.maximum(ssum * dinv[:, None] + b_ref[...], 0.0)
    z = jnp.dot(h, wt_ref[...], preferred_element_type=jnp.float32)
    o_ref[...] = z * dinv[:, None]


_tc2 = pl.pallas_call(
    _tc2_body,
    grid=(N // BN,),
    in_specs=[
        pl.BlockSpec((BN, D), lambda i: (i, 0)),
        pl.BlockSpec((NC, BN, D), lambda i: (0, i, 0)),
        pl.BlockSpec((BN, DINV_W), lambda i: (i, 0)),
        pl.BlockSpec((D, D), lambda i: (0, 0)),
        pl.BlockSpec((1, D), lambda i: (0, 0)),
    ],
    out_specs=pl.BlockSpec((BN, D), lambda i: (i, 0)),
    out_shape=jax.ShapeDtypeStruct((N, D), jnp.float32),
)


def _tc3_body(y_ref, p_ref, dinv16_ref, b_ref, o_ref):
    dinv = _dinv16_of(dinv16_ref[...])
    ssum = p_ref[0] + p_ref[1] + y_ref[...]
    o_ref[...] = jnp.maximum(ssum * dinv[:, None] + b_ref[...], 0.0)


_tc3 = pl.pallas_call(
    _tc3_body,
    grid=(N // BN,),
    in_specs=[
        pl.BlockSpec((BN, D), lambda i: (i, 0)),
        pl.BlockSpec((NC, BN, D), lambda i: (0, i, 0)),
        pl.BlockSpec((BN, DINV_W), lambda i: (i, 0)),
        pl.BlockSpec((1, D), lambda i: (0, 0)),
    ],
    out_specs=pl.BlockSpec((BN, D), lambda i: (i, 0)),
    out_shape=jax.ShapeDtypeStruct((N, D), jnp.float32),
)


def kernel(x, edge_index, W1, b1, W2, b2):
    row = edge_index[0]
    col = edge_index[1]
    z128 = jnp.zeros((NP, D), jnp.float32)
    ones = jnp.ones((EB, DEG_W), jnp.float32)

    deg_k, msg_k = _get_deg2(), _get_msg2()
    degp = deg_k(col, ones, z128)
    z1 = _tc_mm(x, W1.T)
    y1, dinv16 = _tc_scale(z1, degp)
    p1 = msg_k(y1, row, col, z128)
    y2 = _tc2(y1, p1, dinv16, W2.T, b1.reshape(1, D))
    p2 = msg_k(y2, row, col, z128)
    return _tc3(y2, p2, dinv16, b2.reshape(1, D))
